# baseline (speedup 1.0000x reference)
import functools

import jax
import jax.numpy as jnp
from jax import lax
from jax.experimental import pallas as pl
from jax.experimental.pallas import tpu as pltpu
from jax.experimental.pallas import tpu_sc as plsc

N_NODES = 50000
NUM_GRAPHS = 64
C = 16
NPAD = 50176
NCORES = 2
NSUB = 16
NW = NCORES * NSUB
ROWS_PER_SUB = NPAD // NSUB

CHUNK = 128
SUP = 6
NSUP = 66
NPAIR = NSUP // 2
NCH = SUP * NSUP
EDGES_PAD = NW * NCH * CHUNK
SUPE = SUP * CHUNK

BR = 6272
GSTEPS = NPAD // BR
EPS = 1e-5

POOL_CHUNK = 112
NPCH = (NPAD // NW) // POOL_CHUNK

def _mesh():
    return plsc.VectorSubcoreMesh(core_axis_name="c", subcore_axis_name="s",
                                  num_cores=NCORES, num_subcores=NSUB)


@functools.cache
def _make_spmm():
  @functools.partial(
      pl.kernel,
      out_type=jax.ShapeDtypeStruct((NCORES, NPAD, C), jnp.float32),
      mesh=_mesh(),
      compiler_params=pltpu.CompilerParams(use_tc_tiling_on_sc=False),
      scratch_types=[
          pltpu.VMEM((2, SUP, CHUNK), jnp.int32),
          pltpu.VMEM((2, SUP, CHUNK), jnp.int32),
          pltpu.VMEM((2, SUPE, C), jnp.float32),
          pltpu.VMEM_SHARED((NPAD, C), jnp.float32),
          pltpu.VMEM_SHARED((NPAD, C), jnp.float32),
          pltpu.SemaphoreType.DMA,
          pltpu.SemaphoreType.DMA,
      ],
  )
  def _spmm(y_hbm, src_hbm, dst_hbm, zero_hbm, out_hbm,
            isrc, idst, rows, acc, ytab, sem0, sem1):
    cid = lax.axis_index("c")
    sid = lax.axis_index("s")
    wid = cid * NSUB + sid
    r0 = sid * ROWS_PER_SUB

    @pl.when(cid == 0)
    def _():
        pltpu.sync_copy(y_hbm.at[pl.ds(r0, ROWS_PER_SUB)],
                        acc.at[pl.ds(r0, ROWS_PER_SUB)])

    @pl.when(cid > 0)
    def _():
        pltpu.sync_copy(zero_hbm.at[pl.ds(r0, ROWS_PER_SUB)],
                        acc.at[pl.ds(r0, ROWS_PER_SUB)])

    pltpu.sync_copy(y_hbm.at[pl.ds(r0, ROWS_PER_SUB)],
                    ytab.at[pl.ds(r0, ROWS_PER_SUB)])
    plsc.subcore_barrier()

    cbase = wid * NSUP

    def fire(s, buf, ib, db, rb, sem):
        cb = cbase + s
        pltpu.sync_copy(src_hbm.at[cb], ib)
        pltpu.sync_copy(dst_hbm.at[cb], db)
        for j in range(SUP):
            pltpu.async_copy(ytab.at[ib.at[j]],
                             rb.at[pl.ds(j * CHUNK, CHUNK)], sem)

    def drain_scatter(db, rb, sem):
        pltpu.make_async_copy(y_hbm.at[pl.ds(0, SUPE)], rb, sem).wait()
        for j in range(SUP):
            pltpu.sync_copy(rb.at[pl.ds(j * CHUNK, CHUNK)],
                            acc.at[db.at[j]], add=True)

    fire(0, 0, isrc.at[0], idst.at[0], rows.at[0], sem0)

    def pair(p, carry):
        s0 = 2 * p
        fire(s0 + 1, 1, isrc.at[1], idst.at[1], rows.at[1], sem1)
        drain_scatter(idst.at[0], rows.at[0], sem0)

        @pl.when(p + 1 < NPAIR)
        def _():
            fire(s0 + 2, 0, isrc.at[0], idst.at[0], rows.at[0], sem0)

        drain_scatter(idst.at[1], rows.at[1], sem1)
        return carry

    lax.fori_loop(0, NPAIR, pair, 0)

    plsc.subcore_barrier()
    pltpu.sync_copy(acc.at[pl.ds(r0, ROWS_PER_SUB)],
                    out_hbm.at[cid, pl.ds(r0, ROWS_PER_SUB)])

  return _spmm


def _spmm_call(y, src3d, dst3d, zeros):
    return _make_spmm()(y, src3d, dst3d, zeros)


@functools.cache
def _make_pool():
  @functools.partial(
      pl.kernel,
      out_type=jax.ShapeDtypeStruct((NCORES, 2, 128, C), jnp.float32),
      mesh=_mesh(),
      compiler_params=pltpu.CompilerParams(use_tc_tiling_on_sc=False),
      scratch_types=[
          pltpu.VMEM((NPCH, POOL_CHUNK), jnp.int32),
          pltpu.VMEM((POOL_CHUNK, C), jnp.float32),
          pltpu.VMEM((POOL_CHUNK, C), jnp.float32),
          pltpu.VMEM_SHARED((128, C), jnp.float32),
          pltpu.VMEM_SHARED((128, C), jnp.float32),
      ],
  )
  def _pool(h_hbm, bidx_hbm, mask_hbm, zero_hbm, out_hbm,
            bidx, hrows, mrows, accs, accc):
    cid = lax.axis_index("c")
    sid = lax.axis_index("s")
    wid = cid * NSUB + sid
    rowbase = wid * (NPCH * POOL_CHUNK)

    z0 = sid * 8
    pltpu.sync_copy(zero_hbm.at[pl.ds(z0, 8)], accs.at[pl.ds(z0, 8)])
    pltpu.sync_copy(zero_hbm.at[pl.ds(z0, 8)], accc.at[pl.ds(z0, 8)])
    plsc.subcore_barrier()

    pltpu.sync_copy(bidx_hbm.at[wid], bidx)

    def body(j, carry):
        r = rowbase + j * POOL_CHUNK
        pltpu.sync_copy(h_hbm.at[pl.ds(r, POOL_CHUNK)], hrows)
        pltpu.sync_copy(mask_hbm.at[pl.ds(r, POOL_CHUNK)], mrows)
        pltpu.sync_copy(hrows, accs.at[bidx.at[j]], add=True)
        pltpu.sync_copy(mrows, accc.at[bidx.at[j]], add=True)
        return carry

    lax.fori_loop(0, NPCH, body, 0)

    plsc.subcore_barrier()
    pltpu.sync_copy(accs.at[pl.ds(z0, 8)], out_hbm.at[cid, 0, pl.ds(z0, 8)])
    pltpu.sync_copy(accc.at[pl.ds(z0, 8)], out_hbm.at[cid, 1, pl.ds(z0, 8)])

  return _pool


def _pool_call(h, bidx2d, mask, zeros):
    return _make_pool()(h, bidx2d, mask, zeros)


def _rowmask(i, ch):
    rid = lax.broadcasted_iota(jnp.int32, (BR, ch), 0) + i * BR
    return rid < N_NODES


def _rb(ch):
    return pl.BlockSpec((BR, ch), lambda i: (i, 0))


def _full(r, ch):
    return pl.BlockSpec((r, ch), lambda i: (0, 0))


@functools.cache
def _make_prep():
    def body(d0, d1, x, dinv, xt):
        i = pl.program_id(0)
        deg = d0[...] + d1[...]
        dv = jnp.where(_rowmask(i, C),
                       lax.rsqrt(jnp.maximum(deg, 1.0)), 0.0)
        dinv[...] = dv
        xt[...] = dv * x[...]

    return pl.pallas_call(
        body,
        grid=(GSTEPS,),
        in_specs=[_rb(C), _rb(C), _rb(C)],
        out_specs=[_rb(C), _rb(C)],
        out_shape=[jax.ShapeDtypeStruct((NPAD, C), jnp.float32)] * 2,
    )


@functools.cache
def _make_post(cout, with_w):
    def body(*refs):
        if with_w:
            p0, p1, dinv, w, b, c_ref, st_ref = refs
        else:
            p0, p1, dinv, b, c_ref, st_ref = refs
        i = pl.program_id(0)
        agg = dinv[...] * (p0[...] + p1[...])
        if with_w:
            z = jnp.dot(agg, w[...], preferred_element_type=jnp.float32)
        else:
            z = agg
        cval = jnp.where(_rowmask(i, cout), z + b[0:1, :], 0.0)
        c_ref[...] = cval
        s1 = jnp.sum(cval, axis=0, keepdims=True)
        s2 = jnp.sum(cval * cval, axis=0, keepdims=True)
        blk = jnp.concatenate(
            [s1, s2, jnp.zeros((6, cout), jnp.float32)], axis=0)

        @pl.when(i == 0)
        def _():
            st_ref[...] = blk

        @pl.when(i > 0)
        def _():
            st_ref[...] += blk

    in_specs = [_rb(C), _rb(C), _rb(C)]
    if with_w:
        in_specs.append(_full(C, cout))
    in_specs.append(_full(8, cout))
    return pl.pallas_call(
        body,
        grid=(GSTEPS,),
        in_specs=in_specs,
        out_specs=[_rb(cout), _full(8, cout)],
        out_shape=[jax.ShapeDtypeStruct((NPAD, cout), jnp.float32),
                   jax.ShapeDtypeStruct((8, cout), jnp.float32)],
    )


@functools.cache
def _make_apply(ct, cres, has_res, has_proj, emit_h, emit_y):
    def body(*refs):
        refs = list(refs)
        c_ref = refs.pop(0)
        st_ref = refs.pop(0)
        g_ref = refs.pop(0)
        be_ref = refs.pop(0)
        hp_ref = refs.pop(0) if has_res else None
        wp_ref = refs.pop(0) if has_proj else None
        wn_ref = refs.pop(0) if emit_y else None
        dinv_ref = refs.pop(0) if emit_y else None
        h_ref = refs.pop(0) if emit_h else None
        y_ref = refs.pop(0) if emit_y else None

        i = pl.program_id(0)
        st = st_ref[...]
        m = st[0:1, :] * (1.0 / N_NODES)
        ex2 = st[1:2, :] * (1.0 / N_NODES)
        rstd = lax.rsqrt(jnp.maximum(ex2 - m * m, 0.0) + EPS)
        t = (c_ref[...] - m) * (rstd * g_ref[0:1, :]) + be_ref[0:1, :]
        if has_res:
            r = hp_ref[...]
            if has_proj:
                r = jnp.dot(r, wp_ref[...], preferred_element_type=jnp.float32)
            t = t + r
        t = jnp.where(_rowmask(i, ct), jnp.maximum(t, 0.0), 0.0)
        if emit_h:
            h_ref[...] = t
        if emit_y:
            y_ref[...] = dinv_ref[...] * jnp.dot(
                t, wn_ref[...], preferred_element_type=jnp.float32)

    in_specs = [_rb(ct), _full(8, ct), _full(8, ct), _full(8, ct)]
    if has_res:
        in_specs.append(_rb(cres))
    if has_proj:
        in_specs.append(_full(cres, ct))
    if emit_y:
        in_specs.append(_full(ct, C))
        in_specs.append(_rb(C))
    out_specs, out_shape = [], []
    if emit_h:
        out_specs.append(_rb(ct))
        out_shape.append(jax.ShapeDtypeStruct((NPAD, ct), jnp.float32))
    if emit_y:
        out_specs.append(_rb(C))
        out_shape.append(jax.ShapeDtypeStruct((NPAD, C), jnp.float32))
    return pl.pallas_call(
        body,
        grid=(GSTEPS,),
        in_specs=in_specs,
        out_specs=out_specs,
        out_shape=out_shape,
    )


@functools.cache
def _make_pool_fc(ncls):
    def body(s0, s1, c0, c1, w, b, o):
        s = s0[...] + s1[...]
        cnt = c0[...] + c1[...]
        pooled = s[0:NUM_GRAPHS, :] / jnp.maximum(cnt[0:NUM_GRAPHS, :], 1.0)
        o[...] = jnp.dot(pooled, w[...],
                         preferred_element_type=jnp.float32) + b[0:1, :]

    return pl.pallas_call(
        body,
        grid=(1,),
        in_specs=[_full(128, C), _full(128, C), _full(128, C), _full(128, C),
                  _full(C, ncls), _full(8, ncls)],
        out_specs=_full(NUM_GRAPHS, ncls),
        out_shape=jax.ShapeDtypeStruct((NUM_GRAPHS, ncls), jnp.float32),
    )


def _b8(v):
    return jnp.broadcast_to(v.astype(jnp.float32)[None, :], (8, v.shape[0]))


def kernel(x, params, ei, batch):
    n = x.shape[0]
    e = ei.shape[1]
    f32 = jnp.float32

    npad_e = EDGES_PAD - e
    pad_ids = (n + (jnp.arange(npad_e, dtype=jnp.int32) % 128)).astype(jnp.int32)
    src3d = jnp.concatenate([ei[0], pad_ids]).reshape(-1, SUP, CHUNK)
    dst3d = jnp.concatenate([ei[1], pad_ids]).reshape(-1, SUP, CHUNK)

    zeros_t = jnp.zeros((NPAD, C), f32)
    ones_t = jnp.pad(jnp.ones((n, C), f32), ((0, NPAD - n), (0, 0)))
    x16 = jnp.pad(x.astype(f32), ((0, NPAD - n), (0, C - x.shape[1])))

    deg_parts = _spmm_call(ones_t, src3d, dst3d, zeros_t)
    dinv, xt = _make_prep()(deg_parts[0], deg_parts[1], x16)

    p = _spmm_call(xt, src3d, dst3d, zeros_t)
    w1 = jnp.pad(params['conv1_W'].astype(f32),
                 ((0, C - params['conv1_W'].shape[0]), (0, 0)))
    c64 = w1.shape[1]
    cz, st = _make_post(c64, True)(p[0], p[1], dinv, w1, _b8(params['conv1_b']))

    blocks = [blk for layer in params['layers'] for blk in layer]
    w_first = blocks[0]['W1'].astype(f32)
    h, y = _make_apply(c64, 0, False, False, True, True)(
        cz, st, _b8(params['bn1_g']), _b8(params['bn1_b']), w_first, dinv)

    for bi, blk in enumerate(blocks):
        cin = blk['W1'].shape[0]
        p = _spmm_call(y, src3d, dst3d, zeros_t)
        cA, stA = _make_post(C, False)(p[0], p[1], dinv, _b8(blk['b1']))
        (yB,) = _make_apply(C, 0, False, False, False, True)(
            cA, stA, _b8(blk['g1']), _b8(blk['be1']),
            blk['W2'].astype(f32), dinv)
        p = _spmm_call(yB, src3d, dst3d, zeros_t)
        cB, stB = _make_post(C, False)(p[0], p[1], dinv, _b8(blk['b2']))
        has_proj = 'Wp' in blk
        last = bi == len(blocks) - 1
        args = [cB, stB, _b8(blk['g2']), _b8(blk['be2']), h]
        if has_proj:
            args.append(blk['Wp'].astype(f32))
        if not last:
            args.append(blocks[bi + 1]['W1'].astype(f32))
            args.append(dinv)
        outs = _make_apply(C, cin, True, has_proj, True, not last)(*args)
        if last:
            (h,) = outs
        else:
            h, y = outs

    bpad = (NUM_GRAPHS + (jnp.arange(NPAD - n, dtype=jnp.int32) % 32)
            ).astype(jnp.int32)
    bidx3d = jnp.concatenate([batch.astype(jnp.int32), bpad]).reshape(
        NW, NPCH, POOL_CHUNK)
    pool = _pool_call(h, bidx3d, ones_t, zeros_t)
    fcw = params['fc_W'].astype(f32)
    out = _make_pool_fc(fcw.shape[1])(
        pool[0, 0], pool[1, 0], pool[0, 1], pool[1, 1], fcw,
        _b8(params['fc_b']))
    return out


# speedup vs baseline: 1.1868x; 1.1868x over previous
import functools

import jax
import jax.numpy as jnp
from jax import lax
from jax.experimental import pallas as pl
from jax.experimental.pallas import tpu as pltpu
from jax.experimental.pallas import tpu_sc as plsc

N_NODES = 50000
NUM_GRAPHS = 64
C = 16
NPAD = 50176
NCORES = 2
NSUB = 16
NW = NCORES * NSUB
ROWS_PER_SUB = NPAD // NSUB

CHUNK = 128
SUP = 6
NSUP = 66
NPAIR = NSUP // 2
NCH = SUP * NSUP
EDGES_PAD = NW * NCH * CHUNK
SUPE = SUP * CHUNK

NR = NPAD // 8
GSTEPS = 8
BRP = NR // GSTEPS
EPS = 1e-5

POOL_CHUNK = 112
NPCH = (NPAD // NW) // POOL_CHUNK


def _mesh():
    return plsc.VectorSubcoreMesh(core_axis_name="c", subcore_axis_name="s",
                                  num_cores=NCORES, num_subcores=NSUB)


@functools.cache
def _make_spmm():
  @functools.partial(
      pl.kernel,
      out_type=jax.ShapeDtypeStruct((NCORES, NPAD, C), jnp.float32),
      mesh=_mesh(),
      compiler_params=pltpu.CompilerParams(use_tc_tiling_on_sc=False),
      scratch_types=[
          pltpu.VMEM((2, SUP, CHUNK), jnp.int32),
          pltpu.VMEM((2, SUP, CHUNK), jnp.int32),
          pltpu.VMEM((2, SUPE, C), jnp.float32),
          pltpu.VMEM_SHARED((NPAD, C), jnp.float32),
          pltpu.SemaphoreType.DMA,
          pltpu.SemaphoreType.DMA,
      ],
  )
  def _spmm(y_hbm, src_hbm, dst_hbm, zero_hbm, out_hbm,
            isrc, idst, rows, acc, sem0, sem1):
    cid = lax.axis_index("c")
    sid = lax.axis_index("s")
    wid = cid * NSUB + sid
    r0 = sid * ROWS_PER_SUB

    @pl.when(cid == 0)
    def _():
        pltpu.sync_copy(y_hbm.at[pl.ds(r0, ROWS_PER_SUB)],
                        acc.at[pl.ds(r0, ROWS_PER_SUB)])

    @pl.when(cid > 0)
    def _():
        pltpu.sync_copy(zero_hbm.at[pl.ds(r0, ROWS_PER_SUB)],
                        acc.at[pl.ds(r0, ROWS_PER_SUB)])

    plsc.subcore_barrier()

    cbase = wid * NSUP

    def fire(s, buf, ib, db, rb, sem):
        cb = cbase + s
        pltpu.sync_copy(src_hbm.at[cb], ib)
        pltpu.sync_copy(dst_hbm.at[cb], db)
        for j in range(SUP):
            pltpu.async_copy(y_hbm.at[ib.at[j]],
                             rb.at[pl.ds(j * CHUNK, CHUNK)], sem)

    def drain_scatter(db, rb, sem):
        pltpu.make_async_copy(y_hbm.at[pl.ds(0, SUPE)], rb, sem).wait()
        for j in range(SUP):
            pltpu.sync_copy(rb.at[pl.ds(j * CHUNK, CHUNK)],
                            acc.at[db.at[j]], add=True)

    fire(0, 0, isrc.at[0], idst.at[0], rows.at[0], sem0)

    def pair(p, carry):
        s0 = 2 * p
        fire(s0 + 1, 1, isrc.at[1], idst.at[1], rows.at[1], sem1)
        drain_scatter(idst.at[0], rows.at[0], sem0)

        @pl.when(p + 1 < NPAIR)
        def _():
            fire(s0 + 2, 0, isrc.at[0], idst.at[0], rows.at[0], sem0)

        drain_scatter(idst.at[1], rows.at[1], sem1)
        return carry

    lax.fori_loop(0, NPAIR, pair, 0)

    plsc.subcore_barrier()
    pltpu.sync_copy(acc.at[pl.ds(r0, ROWS_PER_SUB)],
                    out_hbm.at[cid, pl.ds(r0, ROWS_PER_SUB)])

  return _spmm


def _spmm_call(y, src3d, dst3d, zeros):
    return _make_spmm()(y, src3d, dst3d, zeros)


@functools.cache
def _make_pool():
  @functools.partial(
      pl.kernel,
      out_type=jax.ShapeDtypeStruct((NCORES, 2, 128, C), jnp.float32),
      mesh=_mesh(),
      compiler_params=pltpu.CompilerParams(use_tc_tiling_on_sc=False),
      scratch_types=[
          pltpu.VMEM((NPCH, POOL_CHUNK), jnp.int32),
          pltpu.VMEM((POOL_CHUNK, C), jnp.float32),
          pltpu.VMEM((POOL_CHUNK, C), jnp.float32),
          pltpu.VMEM_SHARED((128, C), jnp.float32),
          pltpu.VMEM_SHARED((128, C), jnp.float32),
      ],
  )
  def _pool(h_hbm, bidx_hbm, mask_hbm, zero_hbm, out_hbm,
            bidx, hrows, mrows, accs, accc):
    cid = lax.axis_index("c")
    sid = lax.axis_index("s")
    wid = cid * NSUB + sid
    rowbase = wid * (NPCH * POOL_CHUNK)

    z0 = sid * 8
    pltpu.sync_copy(zero_hbm.at[pl.ds(z0, 8)], accs.at[pl.ds(z0, 8)])
    pltpu.sync_copy(zero_hbm.at[pl.ds(z0, 8)], accc.at[pl.ds(z0, 8)])
    plsc.subcore_barrier()

    pltpu.sync_copy(bidx_hbm.at[wid], bidx)

    def body(j, carry):
        r = rowbase + j * POOL_CHUNK
        pltpu.sync_copy(h_hbm.at[pl.ds(r, POOL_CHUNK)], hrows)
        pltpu.sync_copy(mask_hbm.at[pl.ds(r, POOL_CHUNK)], mrows)
        pltpu.sync_copy(hrows, accs.at[bidx.at[j]], add=True)
        pltpu.sync_copy(mrows, accc.at[bidx.at[j]], add=True)
        return carry

    lax.fori_loop(0, NPCH, body, 0)

    plsc.subcore_barrier()
    pltpu.sync_copy(accs.at[pl.ds(z0, 8)], out_hbm.at[cid, 0, pl.ds(z0, 8)])
    pltpu.sync_copy(accc.at[pl.ds(z0, 8)], out_hbm.at[cid, 1, pl.ds(z0, 8)])

  return _pool


def _pool_call(h, bidx3d, mask, zeros):
    return _make_pool()(h, bidx3d, mask, zeros)


def _pmask(i, cw):
    ch = cw // 8
    r = lax.broadcasted_iota(jnp.int32, (BRP, cw), 0) + i * BRP
    ph = lax.broadcasted_iota(jnp.int32, (BRP, cw), 1) // ch
    return (r * 8 + ph) < N_NODES


def _rb(cw):
    return pl.BlockSpec((BRP, cw), lambda i: (i, 0))


def _full(r, cw):
    return pl.BlockSpec((r, cw), lambda i: (0, 0))


@functools.cache
def _make_prep():
    def body(d0, d1, x, dinv, xt):
        i = pl.program_id(0)
        deg = d0[...] + d1[...]
        dv = jnp.where(_pmask(i, 128),
                       lax.rsqrt(jnp.maximum(deg, 1.0)), 0.0)
        dinv[...] = dv
        xt[...] = dv * x[...]

    return pl.pallas_call(
        body,
        grid=(GSTEPS,),
        in_specs=[_rb(128), _rb(128), _rb(128)],
        out_specs=[_rb(128), _rb(128)],
        out_shape=[jax.ShapeDtypeStruct((NR, 128), jnp.float32)] * 2,
    )


@functools.cache
def _make_post(cw_out, with_w):
    def body(*refs):
        if with_w:
            p0, p1, dinv, w, b, c_ref, st_ref = refs
        else:
            p0, p1, dinv, b, c_ref, st_ref = refs
        i = pl.program_id(0)
        agg = dinv[...] * (p0[...] + p1[...])
        if with_w:
            z = jnp.dot(agg, w[...], preferred_element_type=jnp.float32,
                        precision=lax.Precision.HIGHEST)
        else:
            z = agg
        cval = jnp.where(_pmask(i, cw_out), z + b[0:1, :], 0.0)
        c_ref[...] = cval
        s1 = jnp.sum(cval, axis=0, keepdims=True)
        s2 = jnp.sum(cval * cval, axis=0, keepdims=True)
        blk = jnp.concatenate(
            [s1, s2, jnp.zeros((6, cw_out), jnp.float32)], axis=0)

        @pl.when(i == 0)
        def _():
            st_ref[...] = blk

        @pl.when(i > 0)
        def _():
            st_ref[...] += blk

    in_specs = [_rb(128), _rb(128), _rb(128)]
    if with_w:
        in_specs.append(_full(128, cw_out))
    in_specs.append(_full(8, cw_out))
    return pl.pallas_call(
        body,
        grid=(GSTEPS,),
        in_specs=in_specs,
        out_specs=[_rb(cw_out), _full(8, cw_out)],
        out_shape=[jax.ShapeDtypeStruct((NR, cw_out), jnp.float32),
                   jax.ShapeDtypeStruct((8, cw_out), jnp.float32)],
    )


@functools.cache
def _make_apply(cwt, cwres, has_res, has_proj, emit_h, emit_y):
    def body(*refs):
        refs = list(refs)
        c_ref = refs.pop(0)
        st_ref = refs.pop(0)
        mx_ref = refs.pop(0)
        g_ref = refs.pop(0)
        be_ref = refs.pop(0)
        hp_ref = refs.pop(0) if has_res else None
        wp_ref = refs.pop(0) if has_proj else None
        wn_ref = refs.pop(0) if emit_y else None
        dinv_ref = refs.pop(0) if emit_y else None
        h_ref = refs.pop(0) if emit_h else None
        y_ref = refs.pop(0) if emit_y else None

        i = pl.program_id(0)
        tot = jnp.dot(st_ref[0:2, :], mx_ref[...],
                      preferred_element_type=jnp.float32,
                      precision=lax.Precision.HIGHEST)
        m = tot[0:1, :] * (1.0 / N_NODES)
        ex2 = tot[1:2, :] * (1.0 / N_NODES)
        rstd = lax.rsqrt(jnp.maximum(ex2 - m * m, 0.0) + EPS)
        t = (c_ref[...] - m) * (rstd * g_ref[0:1, :]) + be_ref[0:1, :]
        if has_res:
            r = hp_ref[...]
            if has_proj:
                r = jnp.dot(r, wp_ref[...],
                            preferred_element_type=jnp.float32,
                            precision=lax.Precision.HIGHEST)
            t = t + r
        t = jnp.where(_pmask(i, cwt), jnp.maximum(t, 0.0), 0.0)
        if emit_h:
            h_ref[...] = t
        if emit_y:
            y_ref[...] = dinv_ref[...] * jnp.dot(
                t, wn_ref[...], preferred_element_type=jnp.float32,
                precision=lax.Precision.HIGHEST)

    in_specs = [_rb(cwt), _full(8, cwt), _full(cwt, cwt),
                _full(8, cwt), _full(8, cwt)]
    if has_res:
        in_specs.append(_rb(cwres))
    if has_proj:
        in_specs.append(_full(cwres, cwt))
    if emit_y:
        in_specs.append(_full(cwt, 128))
        in_specs.append(_rb(128))
    out_specs, out_shape = [], []
    if emit_h:
        out_specs.append(_rb(cwt))
        out_shape.append(jax.ShapeDtypeStruct((NR, cwt), jnp.float32))
    if emit_y:
        out_specs.append(_rb(128))
        out_shape.append(jax.ShapeDtypeStruct((NR, 128), jnp.float32))
    return pl.pallas_call(
        body,
        grid=(GSTEPS,),
        in_specs=in_specs,
        out_specs=out_specs,
        out_shape=out_shape,
    )


@functools.cache
def _make_pool_fc(ncls):
    def body(s0, s1, c0, c1, w, b, o):
        s = s0[...] + s1[...]
        cnt = c0[...] + c1[...]
        pooled = s[0:NUM_GRAPHS, :] / jnp.maximum(cnt[0:NUM_GRAPHS, :], 1.0)
        o[...] = jnp.dot(pooled, w[...],
                         preferred_element_type=jnp.float32,
                         precision=lax.Precision.HIGHEST) + b[0:1, :]

    return pl.pallas_call(
        body,
        grid=(1,),
        in_specs=[_full(128, C), _full(128, C), _full(128, C), _full(128, C),
                  _full(C, ncls), _full(8, ncls)],
        out_specs=_full(NUM_GRAPHS, ncls),
        out_shape=jax.ShapeDtypeStruct((NUM_GRAPHS, ncls), jnp.float32),
    )


def _t8(v):
    v = v.astype(jnp.float32)
    return jnp.broadcast_to(jnp.tile(v, 8)[None, :], (8, 8 * v.shape[0]))


def _k8(w):
    return jnp.kron(jnp.eye(8, dtype=jnp.float32), w.astype(jnp.float32))


def _b8(v):
    return jnp.broadcast_to(v.astype(jnp.float32)[None, :], (8, v.shape[0]))


def _phase_sum(ch):
    return jnp.kron(jnp.ones((8, 8), jnp.float32),
                    jnp.eye(ch, dtype=jnp.float32))


def kernel(x, params, ei, batch):
    n = x.shape[0]
    e = ei.shape[1]
    f32 = jnp.float32

    npad_e = EDGES_PAD - e
    pad_ids = (n + (jnp.arange(npad_e, dtype=jnp.int32) % 128)).astype(jnp.int32)
    src3d = jnp.concatenate([ei[0], pad_ids]).reshape(-1, SUP, CHUNK)
    dst3d = jnp.concatenate([ei[1], pad_ids]).reshape(-1, SUP, CHUNK)

    zeros_t = jnp.zeros((NPAD, C), f32)
    ones_t = jnp.pad(jnp.ones((n, C), f32), ((0, NPAD - n), (0, 0)))
    x16p = jnp.pad(x.astype(f32),
                   ((0, NPAD - n), (0, C - x.shape[1]))).reshape(NR, 128)

    m16 = _phase_sum(16)

    def packed(a):
        return a.reshape(NR, -1)

    dp = _spmm_call(ones_t, src3d, dst3d, zeros_t)
    dinv, xt = _make_prep()(packed(dp[0]), packed(dp[1]), x16p)

    p = _spmm_call(xt.reshape(NPAD, C), src3d, dst3d, zeros_t)
    w1 = jnp.pad(params['conv1_W'].astype(f32),
                 ((0, C - params['conv1_W'].shape[0]), (0, 0)))
    cw64 = 8 * w1.shape[1]
    cz, st = _make_post(cw64, True)(
        packed(p[0]), packed(p[1]), dinv, _k8(w1), _t8(params['conv1_b']))

    blocks = [blk for layer in params['layers'] for blk in layer]
    m64 = _phase_sum(w1.shape[1])
    h, y = _make_apply(cw64, 0, False, False, True, True)(
        cz, st, m64, _t8(params['bn1_g']), _t8(params['bn1_b']),
        _k8(blocks[0]['W1']), dinv)

    for bi, blk in enumerate(blocks):
        cwin = 8 * blk['W1'].shape[0]
        p = _spmm_call(y.reshape(NPAD, C), src3d, dst3d, zeros_t)
        cA, stA = _make_post(128, False)(
            packed(p[0]), packed(p[1]), dinv, _t8(blk['b1']))
        (yB,) = _make_apply(128, 0, False, False, False, True)(
            cA, stA, m16, _t8(blk['g1']), _t8(blk['be1']),
            _k8(blk['W2']), dinv)
        p = _spmm_call(yB.reshape(NPAD, C), src3d, dst3d, zeros_t)
        cB, stB = _make_post(128, False)(
            packed(p[0]), packed(p[1]), dinv, _t8(blk['b2']))
        has_proj = 'Wp' in blk
        last = bi == len(blocks) - 1
        args = [cB, stB, m16, _t8(blk['g2']), _t8(blk['be2']), h]
        if has_proj:
            args.append(_k8(blk['Wp']))
        if not last:
            args.append(_k8(blocks[bi + 1]['W1']))
            args.append(dinv)
        outs = _make_apply(128, cwin, True, has_proj, True, not last)(*args)
        if last:
            (h,) = outs
        else:
            h, y = outs

    bpad = (NUM_GRAPHS + (jnp.arange(NPAD - n, dtype=jnp.int32) % 32)
            ).astype(jnp.int32)
    bidx3d = jnp.concatenate([batch.astype(jnp.int32), bpad]).reshape(
        NW, NPCH, POOL_CHUNK)
    pool = _pool_call(h.reshape(NPAD, C), bidx3d, ones_t, zeros_t)
    fcw = params['fc_W'].astype(f32)
    out = _make_pool_fc(fcw.shape[1])(
        pool[0, 0], pool[1, 0], pool[0, 1], pool[1, 1], fcw,
        _b8(params['fc_b']))
    return out


# speedup vs baseline: 1.9414x; 1.6359x over previous
import functools

import jax
import jax.numpy as jnp
from jax import lax
from jax.experimental import pallas as pl
from jax.experimental.pallas import tpu as pltpu
from jax.experimental.pallas import tpu_sc as plsc

N_NODES = 50000
NUM_GRAPHS = 64
C = 16
NPAD = 50176
NCORES = 2
NSUB = 16
NW = NCORES * NSUB
ROWS_PER_SUB = NPAD // NSUB

CHUNK = 128
SUP = 6
NSUP = 66
NPAIR = NSUP // 2
NCH = SUP * NSUP
EDGES_PAD = NW * NCH * CHUNK
SUPE = SUP * CHUNK

NR = NPAD // 8
GSTEPS = 8
BRP = NR // GSTEPS
EPS = 1e-5

POOL_CHUNK = 112
NPCH = (NPAD // NW) // POOL_CHUNK


def _mesh():
    return plsc.VectorSubcoreMesh(core_axis_name="c", subcore_axis_name="s",
                                  num_cores=NCORES, num_subcores=NSUB)


@functools.cache
def _make_spmm():
  @functools.partial(
      pl.kernel,
      out_type=jax.ShapeDtypeStruct((NCORES, NPAD, C), jnp.float32),
      mesh=_mesh(),
      compiler_params=pltpu.CompilerParams(use_tc_tiling_on_sc=False),
      scratch_types=[
          pltpu.VMEM((2, SUP, CHUNK), jnp.int32),
          pltpu.VMEM((2, SUP, CHUNK), jnp.int32),
          pltpu.VMEM((2, SUPE, C), jnp.float32),
          pltpu.VMEM_SHARED((NPAD, C), jnp.float32),
          pltpu.SemaphoreType.DMA,
          pltpu.SemaphoreType.DMA,
      ],
  )
  def _spmm(y_hbm, src_hbm, dst_hbm, zero_hbm, out_hbm,
            isrc, idst, rows, acc, sem0, sem1):
    cid = lax.axis_index("c")
    sid = lax.axis_index("s")
    wid = cid * NSUB + sid
    r0 = sid * ROWS_PER_SUB

    @pl.when(cid == 0)
    def _():
        pltpu.sync_copy(y_hbm.at[pl.ds(r0, ROWS_PER_SUB)],
                        acc.at[pl.ds(r0, ROWS_PER_SUB)])

    @pl.when(cid > 0)
    def _():
        pltpu.sync_copy(zero_hbm.at[pl.ds(r0, ROWS_PER_SUB)],
                        acc.at[pl.ds(r0, ROWS_PER_SUB)])

    plsc.subcore_barrier()

    cbase = wid * NSUP

    def fire(s, buf, ib, db, rb, sem):
        cb = cbase + s
        pltpu.sync_copy(src_hbm.at[cb], ib)
        pltpu.sync_copy(dst_hbm.at[cb], db)
        for j in range(SUP):
            pltpu.async_copy(y_hbm.at[ib.at[j]],
                             rb.at[pl.ds(j * CHUNK, CHUNK)], sem)

    def drain_scatter(db, rb, sem):
        pltpu.make_async_copy(y_hbm.at[pl.ds(0, SUPE)], rb, sem).wait()
        for j in range(SUP):
            pltpu.sync_copy(rb.at[pl.ds(j * CHUNK, CHUNK)],
                            acc.at[db.at[j]], add=True)

    fire(0, 0, isrc.at[0], idst.at[0], rows.at[0], sem0)

    def pair(p, carry):
        s0 = 2 * p
        fire(s0 + 1, 1, isrc.at[1], idst.at[1], rows.at[1], sem1)
        drain_scatter(idst.at[0], rows.at[0], sem0)

        @pl.when(p + 1 < NPAIR)
        def _():
            fire(s0 + 2, 0, isrc.at[0], idst.at[0], rows.at[0], sem0)

        drain_scatter(idst.at[1], rows.at[1], sem1)
        return carry

    lax.fori_loop(0, NPAIR, pair, 0)

    plsc.subcore_barrier()
    pltpu.sync_copy(acc.at[pl.ds(r0, ROWS_PER_SUB)],
                    out_hbm.at[cid, pl.ds(r0, ROWS_PER_SUB)])

  return _spmm


def _spmm_call(y, src3d, dst3d, zeros):
    return _make_spmm()(y, src3d, dst3d, zeros)


@functools.cache
def _make_pool():
  @functools.partial(
      pl.kernel,
      out_type=jax.ShapeDtypeStruct((NCORES, 2, 128, C), jnp.float32),
      mesh=_mesh(),
      compiler_params=pltpu.CompilerParams(use_tc_tiling_on_sc=False),
      scratch_types=[
          pltpu.VMEM((NPCH, POOL_CHUNK), jnp.int32),
          pltpu.VMEM((POOL_CHUNK, C), jnp.float32),
          pltpu.VMEM((POOL_CHUNK, C), jnp.float32),
          pltpu.VMEM_SHARED((128, C), jnp.float32),
          pltpu.VMEM_SHARED((128, C), jnp.float32),
      ],
  )
  def _pool(h_hbm, bidx_hbm, mask_hbm, zero_hbm, out_hbm,
            bidx, hrows, mrows, accs, accc):
    cid = lax.axis_index("c")
    sid = lax.axis_index("s")
    wid = cid * NSUB + sid
    rowbase = wid * (NPCH * POOL_CHUNK)

    z0 = sid * 8
    pltpu.sync_copy(zero_hbm.at[pl.ds(z0, 8)], accs.at[pl.ds(z0, 8)])
    pltpu.sync_copy(zero_hbm.at[pl.ds(z0, 8)], accc.at[pl.ds(z0, 8)])
    plsc.subcore_barrier()

    pltpu.sync_copy(bidx_hbm.at[wid], bidx)

    def body(j, carry):
        r = rowbase + j * POOL_CHUNK
        pltpu.sync_copy(h_hbm.at[pl.ds(r, POOL_CHUNK)], hrows)
        pltpu.sync_copy(mask_hbm.at[pl.ds(r, POOL_CHUNK)], mrows)
        pltpu.sync_copy(hrows, accs.at[bidx.at[j]], add=True)
        pltpu.sync_copy(mrows, accc.at[bidx.at[j]], add=True)
        return carry

    lax.fori_loop(0, NPCH, body, 0)

    plsc.subcore_barrier()
    pltpu.sync_copy(accs.at[pl.ds(z0, 8)], out_hbm.at[cid, 0, pl.ds(z0, 8)])
    pltpu.sync_copy(accc.at[pl.ds(z0, 8)], out_hbm.at[cid, 1, pl.ds(z0, 8)])

  return _pool


def _pool_call(h, bidx3d, mask, zeros):
    return _make_pool()(h, bidx3d, mask, zeros)


def _fullmask(cw):
    ch = cw // 8
    r = lax.broadcasted_iota(jnp.int32, (NR, cw), 0)
    ph = lax.broadcasted_iota(jnp.int32, (NR, cw), 1) // ch
    return (r * 8 + ph) < N_NODES


def _full(r, cw):
    return pl.BlockSpec((r, cw), lambda: (0, 0))


@functools.cache
def _make_prep():
    def body(pp, x, dinv, xt):
        deg = pp[0:NR, :] + pp[NR:2 * NR, :]
        dv = jnp.where(_fullmask(128),
                       lax.rsqrt(jnp.maximum(deg, 1.0)), 0.0)
        dinv[...] = dv
        xt[...] = dv * x[...]

    return pl.pallas_call(
        body,
        in_specs=[_full(2 * NR, 128), _full(NR, 128)],
        out_specs=[_full(NR, 128), _full(NR, 128)],
        out_shape=[jax.ShapeDtypeStruct((NR, 128), jnp.float32)] * 2,
    )


@functools.cache
def _make_conv(cw_out, with_w, has_res, has_proj, cwres, emit_h, emit_y):
    hi = lax.Precision.HIGHEST

    def body(*refs):
        refs = list(refs)
        pp = refs.pop(0)
        dinv = refs.pop(0)
        w = refs.pop(0) if with_w else None
        b = refs.pop(0)
        mx = refs.pop(0)
        g = refs.pop(0)
        be = refs.pop(0)
        hp = refs.pop(0) if has_res else None
        wp = refs.pop(0) if has_proj else None
        wn = refs.pop(0) if emit_y else None
        h_ref = refs.pop(0) if emit_h else None
        y_ref = refs.pop(0) if emit_y else None

        agg = dinv[...] * (pp[0:NR, :] + pp[NR:2 * NR, :])
        if with_w:
            z = jnp.dot(agg, w[...], preferred_element_type=jnp.float32,
                        precision=hi)
        else:
            z = agg
        mask = _fullmask(cw_out)
        c = jnp.where(mask, z + b[0:1, :], 0.0)
        s1 = jnp.sum(c, axis=0, keepdims=True)
        s2 = jnp.sum(c * c, axis=0, keepdims=True)
        tot = jnp.dot(jnp.concatenate([s1, s2], axis=0), mx[...],
                      preferred_element_type=jnp.float32, precision=hi)
        m = tot[0:1, :] * (1.0 / N_NODES)
        ex2 = tot[1:2, :] * (1.0 / N_NODES)
        rstd = lax.rsqrt(jnp.maximum(ex2 - m * m, 0.0) + EPS)
        t = (c - m) * (rstd * g[0:1, :]) + be[0:1, :]
        if has_res:
            r = hp[...]
            if has_proj:
                r = jnp.dot(r, wp[...], preferred_element_type=jnp.float32,
                            precision=hi)
            t = t + r
        t = jnp.where(mask, jnp.maximum(t, 0.0), 0.0)
        if emit_h:
            h_ref[...] = t
        if emit_y:
            y_ref[...] = dinv[...] * jnp.dot(
                t, wn[...], preferred_element_type=jnp.float32, precision=hi)

    in_specs = [_full(2 * NR, 128), _full(NR, 128)]
    if with_w:
        in_specs.append(_full(128, cw_out))
    in_specs += [_full(8, cw_out), _full(cw_out, cw_out),
                 _full(8, cw_out), _full(8, cw_out)]
    if has_res:
        in_specs.append(_full(NR, cwres))
    if has_proj:
        in_specs.append(_full(cwres, cw_out))
    if emit_y:
        in_specs.append(_full(cw_out, 128))
    out_specs, out_shape = [], []
    if emit_h:
        out_specs.append(_full(NR, cw_out))
        out_shape.append(jax.ShapeDtypeStruct((NR, cw_out), jnp.float32))
    if emit_y:
        out_specs.append(_full(NR, 128))
        out_shape.append(jax.ShapeDtypeStruct((NR, 128), jnp.float32))
    return pl.pallas_call(
        body,
        in_specs=in_specs,
        out_specs=out_specs,
        out_shape=out_shape,
    )


@functools.cache
def _make_pool_fc(ncls):
    def body(s0, s1, c0, c1, w, b, o):
        s = s0[...] + s1[...]
        cnt = c0[...] + c1[...]
        pooled = s[0:NUM_GRAPHS, :] / jnp.maximum(cnt[0:NUM_GRAPHS, :], 1.0)
        o[...] = jnp.dot(pooled, w[...],
                         preferred_element_type=jnp.float32,
                         precision=lax.Precision.HIGHEST) + b[0:1, :]

    return pl.pallas_call(
        body,
        in_specs=[_full(128, C), _full(128, C), _full(128, C), _full(128, C),
                  _full(C, ncls), _full(8, ncls)],
        out_specs=_full(NUM_GRAPHS, ncls),
        out_shape=jax.ShapeDtypeStruct((NUM_GRAPHS, ncls), jnp.float32),
    )


def _t8(v):
    v = v.astype(jnp.float32)
    return jnp.broadcast_to(jnp.tile(v, 8)[None, :], (8, 8 * v.shape[0]))


def _k8(w):
    return jnp.kron(jnp.eye(8, dtype=jnp.float32), w.astype(jnp.float32))


def _b8(v):
    return jnp.broadcast_to(v.astype(jnp.float32)[None, :], (8, v.shape[0]))


def _phase_sum(ch):
    return jnp.kron(jnp.ones((8, 8), jnp.float32),
                    jnp.eye(ch, dtype=jnp.float32))


def kernel(x, params, ei, batch):
    n = x.shape[0]
    e = ei.shape[1]
    f32 = jnp.float32

    npad_e = EDGES_PAD - e
    pad_ids = (n + (jnp.arange(npad_e, dtype=jnp.int32) % 128)).astype(jnp.int32)
    src3d = jnp.concatenate([ei[0], pad_ids]).reshape(-1, SUP, CHUNK)
    dst3d = jnp.concatenate([ei[1], pad_ids]).reshape(-1, SUP, CHUNK)

    zeros_t = jnp.zeros((NPAD, C), f32)
    ones_t = jnp.pad(jnp.ones((n, C), f32), ((0, NPAD - n), (0, 0)))
    x16p = jnp.pad(x.astype(f32),
                   ((0, NPAD - n), (0, C - x.shape[1]))).reshape(NR, 128)

    m16 = _phase_sum(16)

    def stacked(p):
        return p.reshape(2 * NR, 128)

    dp = _spmm_call(ones_t, src3d, dst3d, zeros_t)
    dinv, xt = _make_prep()(stacked(dp), x16p)

    p = _spmm_call(xt.reshape(NPAD, C), src3d, dst3d, zeros_t)
    w1 = jnp.pad(params['conv1_W'].astype(f32),
                 ((0, C - params['conv1_W'].shape[0]), (0, 0)))
    cw64 = 8 * w1.shape[1]
    blocks = [blk for layer in params['layers'] for blk in layer]
    m64 = _phase_sum(w1.shape[1])
    h, y = _make_conv(cw64, True, False, False, 0, True, True)(
        stacked(p), dinv, _k8(w1), _t8(params['conv1_b']), m64,
        _t8(params['bn1_g']), _t8(params['bn1_b']), _k8(blocks[0]['W1']))

    for bi, blk in enumerate(blocks):
        cwin = 8 * blk['W1'].shape[0]
        p = _spmm_call(y.reshape(NPAD, C), src3d, dst3d, zeros_t)
        (yB,) = _make_conv(128, False, False, False, 0, False, True)(
            stacked(p), dinv, _t8(blk['b1']), m16,
            _t8(blk['g1']), _t8(blk['be1']), _k8(blk['W2']))
        p = _spmm_call(yB.reshape(NPAD, C), src3d, dst3d, zeros_t)
        has_proj = 'Wp' in blk
        last = bi == len(blocks) - 1
        args = [stacked(p), dinv, _t8(blk['b2']), m16,
                _t8(blk['g2']), _t8(blk['be2']), h]
        if has_proj:
            args.append(_k8(blk['Wp']))
        if not last:
            args.append(_k8(blocks[bi + 1]['W1']))
        outs = _make_conv(128, False, True, has_proj, cwin,
                          True, not last)(*args)
        if last:
            (h,) = outs
        else:
            h, y = outs

    bpad = (NUM_GRAPHS + (jnp.arange(NPAD - n, dtype=jnp.int32) % 32)
            ).astype(jnp.int32)
    bidx3d = jnp.concatenate([batch.astype(jnp.int32), bpad]).reshape(
        NW, NPCH, POOL_CHUNK)
    pool = _pool_call(h.reshape(NPAD, C), bidx3d, ones_t, zeros_t)
    fcw = params['fc_W'].astype(f32)
    out = _make_pool_fc(fcw.shape[1])(
        pool[0, 0], pool[1, 0], pool[0, 1], pool[1, 1], fcw,
        _b8(params['fc_b']))
    return out
